# baseline (speedup 1.0000x reference)
import jax
import jax.numpy as jnp
from jax.experimental import pallas as pl
from jax.experimental.pallas import tpu as pltpu

_TOKENS = 32768
_HIDDEN = 4096
_EXPERTS = 64
_BT = 512


def _router_block(hs_ref, wt_ref, probs_ref, sel_ref):
    logits = jnp.dot(hs_ref[:], wt_ref[:], preferred_element_type=jnp.float32)
    m = jnp.max(logits, axis=-1, keepdims=True)
    e = jnp.exp(logits - m)
    probs = e / jnp.sum(e, axis=-1, keepdims=True)
    probs_ref[:] = probs
    pm = jnp.max(probs, axis=-1, keepdims=True)
    idx = jax.lax.broadcasted_iota(jnp.int32, probs.shape, 1)
    sel = jnp.min(jnp.where(probs == pm, idx, _EXPERTS), axis=-1)
    sel_ref[0, 0, :] = sel


def kernel(hidden_states, W):
    nb = _TOKENS // _BT
    wt = W.T
    probs, sel = pl.pallas_call(
        _router_block,
        grid=(nb,),
        in_specs=[
            pl.BlockSpec((_BT, _HIDDEN), lambda i: (i, 0)),
            pl.BlockSpec((_HIDDEN, _EXPERTS), lambda i: (0, 0)),
        ],
        out_specs=[
            pl.BlockSpec((_BT, _EXPERTS), lambda i: (i, 0)),
            pl.BlockSpec((1, 1, _BT), lambda i: (i, 0, 0)),
        ],
        out_shape=[
            jax.ShapeDtypeStruct((_TOKENS, _EXPERTS), jnp.float32),
            jax.ShapeDtypeStruct((nb, 1, _BT), jnp.int32),
        ],
        compiler_params=pltpu.CompilerParams(
            dimension_semantics=("arbitrary",),
        ),
    )(hidden_states, wt)
    return probs, sel.reshape(_TOKENS)


# speedup vs baseline: 1.0889x; 1.0889x over previous
import jax
import jax.numpy as jnp
from jax.experimental import pallas as pl
from jax.experimental.pallas import tpu as pltpu

_TOKENS = 32768
_HIDDEN = 4096
_EXPERTS = 64
_BT = 1024


def _router_block(hs_ref, wt_ref, probs_ref, sel_ref):
    logits = jnp.dot(hs_ref[:], wt_ref[:], preferred_element_type=jnp.float32)
    m = jnp.max(logits, axis=-1, keepdims=True)
    e = jnp.exp(logits - m)
    probs = e / jnp.sum(e, axis=-1, keepdims=True)
    probs_ref[:] = probs
    pm = jnp.max(probs, axis=-1, keepdims=True)
    idx = jax.lax.broadcasted_iota(jnp.int32, probs.shape, 1)
    sel = jnp.min(jnp.where(probs == pm, idx, _EXPERTS), axis=-1)
    sel_ref[0, 0, :] = sel


def kernel(hidden_states, W):
    nb = _TOKENS // _BT
    wt = W.T
    probs, sel = pl.pallas_call(
        _router_block,
        grid=(nb,),
        in_specs=[
            pl.BlockSpec((_BT, _HIDDEN), lambda i: (i, 0)),
            pl.BlockSpec((_HIDDEN, _EXPERTS), lambda i: (0, 0)),
        ],
        out_specs=[
            pl.BlockSpec((_BT, _EXPERTS), lambda i: (i, 0)),
            pl.BlockSpec((1, 1, _BT), lambda i: (i, 0, 0)),
        ],
        out_shape=[
            jax.ShapeDtypeStruct((_TOKENS, _EXPERTS), jnp.float32),
            jax.ShapeDtypeStruct((nb, 1, _BT), jnp.int32),
        ],
        compiler_params=pltpu.CompilerParams(
            dimension_semantics=("arbitrary",),
        ),
    )(hidden_states, wt)
    return probs, sel.reshape(_TOKENS)


# speedup vs baseline: 1.0907x; 1.0016x over previous
import jax
import jax.numpy as jnp
from jax.experimental import pallas as pl
from jax.experimental.pallas import tpu as pltpu

_TOKENS = 32768
_HIDDEN = 4096
_EXPERTS = 64
_BT = 1024


def _router_block(hs_ref, wt_ref, probs_ref, sel_ref):
    logits = jnp.dot(hs_ref[:], wt_ref[:], preferred_element_type=jnp.float32)
    m = jnp.max(logits, axis=-1, keepdims=True)
    e = jnp.exp(logits - m)
    probs = e / jnp.sum(e, axis=-1, keepdims=True)
    probs_ref[:] = probs
    pm = jnp.max(probs, axis=-1, keepdims=True)
    idx = jax.lax.broadcasted_iota(jnp.int32, probs.shape, 1)
    sel = jnp.min(jnp.where(probs == pm, idx, _EXPERTS), axis=-1)
    sel_ref[0, 0, :] = sel


def kernel(hidden_states, W):
    nb = _TOKENS // _BT
    wt = W.T
    probs, sel = pl.pallas_call(
        _router_block,
        grid=(nb,),
        in_specs=[
            pl.BlockSpec((_BT, _HIDDEN), lambda i: (i, 0)),
            pl.BlockSpec((_HIDDEN, _EXPERTS), lambda i: (0, 0)),
        ],
        out_specs=[
            pl.BlockSpec((_BT, _EXPERTS), lambda i: (i, 0)),
            pl.BlockSpec((1, 1, _BT), lambda i: (i, 0, 0)),
        ],
        out_shape=[
            jax.ShapeDtypeStruct((_TOKENS, _EXPERTS), jnp.float32),
            jax.ShapeDtypeStruct((nb, 1, _BT), jnp.int32),
        ],
        compiler_params=pltpu.CompilerParams(
            dimension_semantics=("parallel",),
        ),
    )(hidden_states, wt)
    return probs, sel.reshape(_TOKENS)


# speedup vs baseline: 1.1132x; 1.0207x over previous
import jax
import jax.numpy as jnp
from jax.experimental import pallas as pl
from jax.experimental.pallas import tpu as pltpu

_TOKENS = 32768
_HIDDEN = 4096
_EXPERTS = 64
_BT = 1024


def _router_block(hs_ref, wt_ref, probs_ref, sel_ref):
    logits = jax.lax.dot_general(
        hs_ref[:], wt_ref[:], (((1,), (1,)), ((), ())),
        preferred_element_type=jnp.float32)
    m = jnp.max(logits, axis=-1, keepdims=True)
    e = jnp.exp(logits - m)
    probs = e / jnp.sum(e, axis=-1, keepdims=True)
    probs_ref[:] = probs
    pm = jnp.max(probs, axis=-1, keepdims=True)
    idx = jax.lax.broadcasted_iota(jnp.int32, probs.shape, 1)
    sel = jnp.min(jnp.where(probs == pm, idx, _EXPERTS), axis=-1)
    sel_ref[0, 0, :] = sel


def kernel(hidden_states, W):
    nb = _TOKENS // _BT
    probs, sel = pl.pallas_call(
        _router_block,
        grid=(nb,),
        in_specs=[
            pl.BlockSpec((_BT, _HIDDEN), lambda i: (i, 0)),
            pl.BlockSpec((_EXPERTS, _HIDDEN), lambda i: (0, 0)),
        ],
        out_specs=[
            pl.BlockSpec((_BT, _EXPERTS), lambda i: (i, 0)),
            pl.BlockSpec((1, 1, _BT), lambda i: (i, 0, 0)),
        ],
        out_shape=[
            jax.ShapeDtypeStruct((_TOKENS, _EXPERTS), jnp.float32),
            jax.ShapeDtypeStruct((nb, 1, _BT), jnp.int32),
        ],
        compiler_params=pltpu.CompilerParams(
            dimension_semantics=("parallel",),
        ),
    )(hidden_states, W)
    return probs, sel.reshape(_TOKENS)
